# Initial kernel scaffold; baseline (speedup 1.0000x reference)
import jax
import jax.numpy as jnp
from jax.experimental import pallas as pl


def kernel(x, train, bW0, bb0, bW1, bb1, bW2, bb2, emb, tW0, tb0, tW1, tb1, tW2, tb2, tW3, tb3, tW4, tb4):
    raise NotImplementedError("write your pallas kernel here")


# speedup vs baseline: 9.6325x; 9.6325x over previous
import functools

import jax
import jax.numpy as jnp
import numpy as np
from jax.experimental import pallas as pl
from jax.experimental.pallas import tpu as pltpu
from jax.experimental.pallas import tpu_sc as plsc

_ND = 13
_ED = 32
_B = 4096
_NC = 26
_NF = _NC + 1
_NIDX = _B * _NC
_GW = 128
_BM = 512

_VOCAB = 100000
_OFFSETS = np.arange(_NC, dtype=np.int32) * _VOCAB

_KMAP = np.zeros((_NF * _NF,), np.int32)
_TRIMASK = np.zeros((_NF * _NF, 1), np.float32)
_k = 0
for _i in range(_NF):
    for _j in range(_i, _NF):
        _KMAP[_i * _NF + _j] = _k
        _TRIMASK[_i * _NF + _j, 0] = 1.0
        _k += 1


_NW = 32
_CHUNKS = _NIDX // (_NW * _GW)


def _sc_gather(emb, idx):
    mesh = plsc.VectorSubcoreMesh(core_axis_name="c", subcore_axis_name="s")

    @functools.partial(
        pl.kernel,
        out_type=jax.ShapeDtypeStruct((_NIDX, _ED), emb.dtype),
        mesh=mesh,
        scratch_types=[
            pltpu.VMEM((_CHUNKS, _GW), jnp.int32),
            pltpu.VMEM((_GW, _ED), jnp.float32),
            pltpu.SemaphoreType.DMA,
        ],
    )
    def k(emb_hbm, i_hbm, o_hbm, idx_v, rows_v, sem):
        wid = jax.lax.axis_index("s") * 2 + jax.lax.axis_index("c")
        base = wid * (_CHUNKS * _GW)
        pltpu.sync_copy(i_hbm.at[wid], idx_v)

        @pl.loop(0, _CHUNKS)
        def _(j):
            pltpu.async_copy(emb_hbm.at[idx_v.at[j]], rows_v, sem).wait()
            pltpu.sync_copy(rows_v, o_hbm.at[pl.ds(base + j * _GW, _GW)])

    return k(emb, idx)


def _tc_body(x_ref, ef_ref, bW0_ref, bb0_ref, bW1_ref, bb1_ref, bW2_ref,
             bb2_ref, W0d_ref, W0x_ref, tb0_ref, tW1_ref, tb1_ref, tW2_ref,
             tb2_ref, tW3_ref, tb3_ref, tW4_ref, tb4_ref, o_ref):
    f32 = jnp.float32
    dense = x_ref[:, :_ND]
    h = jnp.maximum(jnp.dot(dense, bW0_ref[...], preferred_element_type=f32)
                    + bb0_ref[...], 0.0)
    h = jnp.maximum(jnp.dot(h, bW1_ref[...], preferred_element_type=f32)
                    + bb1_ref[...], 0.0)
    bot = jnp.maximum(jnp.dot(h, bW2_ref[...], preferred_element_type=f32)
                      + bb2_ref[...], 0.0)

    ef = ef_ref[...]
    fs2 = jnp.concatenate([bot, ef], axis=1)
    fs3 = fs2.reshape(_BM, _NF, _ED)
    xa = jax.lax.dot_general(
        fs3, fs3,
        dimension_numbers=(((2,), (2,)), ((0,), (0,))),
        preferred_element_type=f32,
    )
    xa2 = xa.reshape(_BM, _NF * _NF)

    t = (jnp.dot(bot, W0d_ref[...], preferred_element_type=f32)
         + jnp.dot(xa2, W0x_ref[...], preferred_element_type=f32)
         + tb0_ref[...])
    t = jnp.maximum(t, 0.0)
    t = jnp.maximum(jnp.dot(t, tW1_ref[...], preferred_element_type=f32)
                    + tb1_ref[...], 0.0)
    t = jnp.maximum(jnp.dot(t, tW2_ref[...], preferred_element_type=f32)
                    + tb2_ref[...], 0.0)
    t = jnp.maximum(jnp.dot(t, tW3_ref[...], preferred_element_type=f32)
                    + tb3_ref[...], 0.0)
    o_ref[...] = (jnp.dot(t, tW4_ref[...], preferred_element_type=f32)
                  + tb4_ref[...])


def _dense_stages(x, ef2, bW0, bb0, bW1, bb1, bW2, bb2, W0d, W0x, tb0,
                  tW1, tb1, tW2, tb2, tW3, tb3, tW4, tb4):
    grid = (_B // _BM,)
    full = lambda s: pl.BlockSpec(s, lambda i: (0,) * len(s))
    in_specs = [
        pl.BlockSpec((_BM, x.shape[1]), lambda i: (i, 0)),
        pl.BlockSpec((_BM, _NC * _ED), lambda i: (i, 0)),
        full(bW0.shape), full((1, bb0.shape[-1])),
        full(bW1.shape), full((1, bb1.shape[-1])),
        full(bW2.shape), full((1, bb2.shape[-1])),
        full(W0d.shape), full(W0x.shape), full((1, tb0.shape[-1])),
        full(tW1.shape), full((1, tb1.shape[-1])),
        full(tW2.shape), full((1, tb2.shape[-1])),
        full(tW3.shape), full((1, tb3.shape[-1])),
        full(tW4.shape), full((1, tb4.shape[-1])),
    ]
    out_spec = pl.BlockSpec((_BM, 1), lambda i: (i, 0))
    return pl.pallas_call(
        _tc_body,
        grid=grid,
        in_specs=in_specs,
        out_specs=out_spec,
        out_shape=jax.ShapeDtypeStruct((_B, 1), jnp.float32),
    )(x, ef2, bW0, bb0.reshape(1, -1), bW1, bb1.reshape(1, -1),
      bW2, bb2.reshape(1, -1), W0d, W0x, tb0.reshape(1, -1),
      tW1, tb1.reshape(1, -1), tW2, tb2.reshape(1, -1),
      tW3, tb3.reshape(1, -1), tW4, tb4.reshape(1, -1))


def kernel(x, train, bW0, bb0, bW1, bb1, bW2, bb2, emb, tW0, tb0, tW1, tb1,
           tW2, tb2, tW3, tb3, tW4, tb4):
    del train
    cat = x[:, _ND:].astype(jnp.int32) + jnp.asarray(_OFFSETS)[None, :]
    idx = cat.reshape(_NW, _CHUNKS, _GW)

    W0d = tW0[:_ED]
    W0x = tW0[_ED:][jnp.asarray(_KMAP)] * jnp.asarray(_TRIMASK)

    ef = jnp.take(emb, idx.reshape(-1), axis=0)
    ef2 = ef.reshape(_B, _NC * _ED)

    return _dense_stages(x, ef2, bW0, bb0, bW1, bb1, bW2, bb2, W0d, W0x,
                         tb0, tW1, tb1, tW2, tb2, tW3, tb3, tW4, tb4)
